# baseline (speedup 1.0000x reference)
import functools

import jax
import jax.numpy as jnp
from jax import lax
from jax.experimental import pallas as pl
from jax.experimental.pallas import tpu as pltpu
from jax.experimental.pallas import tpu_sc as plsc

N1 = 8192
N2 = 8192
BM = 2048
CW = 1024


def _chamfer_kernel(fixed_ref, movT_ref, mataug_ref, out_ref,
                    rhs_scr, colmin_scr, rowsum_scr):
    i = pl.program_id(0)
    nsteps = pl.num_programs(0)

    @pl.when(i == 0)
    def _init():
        yT = jnp.dot(mataug_ref[...], movT_ref[...],
                     preferred_element_type=jnp.float32)
        rhs_scr[0:3, :] = yT
        rhs_scr[3:4, :] = jnp.sum(yT * yT, axis=0, keepdims=True)
        rhs_scr[4:5, :] = jnp.ones((1, N2), jnp.float32)
        colmin_scr[...] = jnp.full_like(colmin_scr, jnp.inf)
        rowsum_scr[...] = jnp.zeros_like(rowsum_scr)

    xb = fixed_ref[...]
    x2 = jnp.sum(xb * xb, axis=1, keepdims=True)
    lhs = jnp.concatenate(
        [xb * -2.0, jnp.ones((BM, 1), jnp.float32), x2], axis=1)
    row_min = None
    for c in range(N2 // CW):
        d2 = jnp.dot(lhs, rhs_scr[0:5, c * CW:(c + 1) * CW],
                     preferred_element_type=jnp.float32)
        rm = jnp.min(d2, axis=1)
        row_min = rm if row_min is None else jnp.minimum(row_min, rm)
        col_min = jnp.min(d2, axis=0, keepdims=True)
        colmin_scr[0:1, c * CW:(c + 1) * CW] = jnp.minimum(
            colmin_scr[0:1, c * CW:(c + 1) * CW], col_min)

    row_min = jnp.maximum(row_min, 0.0)
    rowsum_scr[...] += jnp.sum(row_min).reshape(1, 1)

    @pl.when(i == nsteps - 1)
    def _fin():
        col_sum = jnp.sum(jnp.maximum(colmin_scr[...], 0.0))
        out_ref[...] = rowsum_scr[...] / N1 + col_sum.reshape(1, 1) / N2


@jax.jit
def _chamfer(fixed_verts, movT_aug, mat_aug):
    grid = N1 // BM
    out = pl.pallas_call(
        _chamfer_kernel,
        grid=(grid,),
        in_specs=[
            pl.BlockSpec((BM, 3), lambda i: (i, 0)),
            pl.BlockSpec((4, N2), lambda i: (0, 0)),
            pl.BlockSpec((3, 4), lambda i: (0, 0)),
        ],
        out_specs=pl.BlockSpec((1, 1), lambda i: (0, 0)),
        out_shape=jax.ShapeDtypeStruct((1, 1), jnp.float32),
        scratch_shapes=[
            pltpu.VMEM((8, N2), jnp.float32),
            pltpu.VMEM((1, N2), jnp.float32),
            pltpu.VMEM((1, 1), jnp.float32),
        ],
    )(fixed_verts, movT_aug, mat_aug)
    return out[0, 0]


NBINS = 64


def _bin_positions(p, n):
    lo = jnp.min(p)
    hi = jnp.max(p)
    scale = NBINS / (hi - lo + 1e-12)
    b = jnp.clip(jnp.floor((p - lo) * scale), 0, NBINS - 1).astype(jnp.int32)
    onehotT = (jnp.arange(NBINS, dtype=jnp.int32)[:, None] == b[None, :]
               ).astype(jnp.float32)
    cumT = jnp.cumsum(onehotT, axis=1)
    counts = cumT[:, -1]
    offsets = jnp.concatenate(
        [jnp.zeros((1,), jnp.float32), jnp.cumsum(counts)[:-1]])
    pos = jnp.sum(onehotT * (cumT - 1.0 + offsets[:, None]), axis=0)
    return pos.astype(jnp.int32)


_NC = 1
_NS = 16
_NW = _NC * _NS
_BP = N1 // _NW
_IC = 128


def _sc_permute_two(planes_a, pos_a, planes_b, pos_b):
    mesh = plsc.VectorSubcoreMesh(core_axis_name="c", subcore_axis_name="s",
                                  num_cores=_NC)

    @functools.partial(
        pl.kernel, mesh=mesh,
        out_type=[jax.ShapeDtypeStruct((N1,), jnp.float32)] * 6,
        scratch_types=[
            pltpu.VMEM((_BP,), jnp.int32),
            pltpu.VMEM((_BP,), jnp.float32),
            pltpu.VMEM((_BP,), jnp.float32),
            pltpu.VMEM((_BP,), jnp.float32),
            pltpu.VMEM((_BP,), jnp.float32),
            pltpu.VMEM_SHARED((N1,), jnp.float32),
            pltpu.VMEM_SHARED((N1,), jnp.float32),
            pltpu.VMEM_SHARED((N1,), jnp.float32),
        ],
    )
    def k(xa_hbm, ya_hbm, za_hbm, posa_hbm, xb_hbm, yb_hbm, zb_hbm, posb_hbm,
          oxa_hbm, oya_hbm, oza_hbm, oxb_hbm, oyb_hbm, ozb_hbm,
          idx_v, xv, yv, zv, zero_v, x_sh, y_sh, z_sh):
        wid = lax.axis_index("s") * _NC + lax.axis_index("c")
        base = wid * _BP
        win = pl.ds(base, _BP)
        L = 16
        for c in range(_BP // L):
            zero_v[pl.ds(c * L, L)] = jnp.zeros((L,), jnp.float32)

        def permute_one(x_hbm, y_hbm, z_hbm, pos_hbm, ox_hbm, oy_hbm, oz_hbm):
            pltpu.sync_copy(zero_v, x_sh.at[win])
            pltpu.sync_copy(zero_v, y_sh.at[win])
            pltpu.sync_copy(zero_v, z_sh.at[win])
            pltpu.sync_copy(pos_hbm.at[win], idx_v)
            pltpu.sync_copy(x_hbm.at[win], xv)
            pltpu.sync_copy(y_hbm.at[win], yv)
            pltpu.sync_copy(z_hbm.at[win], zv)
            plsc.subcore_barrier()
            pltpu.sync_copy(xv, x_sh.at[idx_v], add=True)
            pltpu.sync_copy(yv, y_sh.at[idx_v], add=True)
            pltpu.sync_copy(zv, z_sh.at[idx_v], add=True)
            plsc.subcore_barrier()
            pltpu.sync_copy(x_sh.at[win], ox_hbm.at[win])
            pltpu.sync_copy(y_sh.at[win], oy_hbm.at[win])
            pltpu.sync_copy(z_sh.at[win], oz_hbm.at[win])

        permute_one(xa_hbm, ya_hbm, za_hbm, posa_hbm, oxa_hbm, oya_hbm,
                    oza_hbm)
        plsc.subcore_barrier()
        permute_one(xb_hbm, yb_hbm, zb_hbm, posb_hbm, oxb_hbm, oyb_hbm,
                    ozb_hbm)

    return k(planes_a[0], planes_a[1], planes_a[2], pos_a,
             planes_b[0], planes_b[1], planes_b[2], pos_b)


def kernel(fixed_verts, moving_verts, mat, trans):
    p_fix = fixed_verts[:, 0]
    p_mov = moving_verts @ mat[0, :, 0]
    pos_f = _bin_positions(p_fix, N1)
    pos_m = _bin_positions(p_mov, N2)
    fx, fy, fz, mx, my, mz = _sc_permute_two(
        (fixed_verts[:, 0], fixed_verts[:, 1], fixed_verts[:, 2]), pos_f,
        (moving_verts[:, 0], moving_verts[:, 1], moving_verts[:, 2]), pos_m)
    fixed_verts = jnp.stack([fx, fy, fz], axis=1)
    movT_aug = jnp.stack(
        [mx, my, mz, jnp.ones((N2,), jnp.float32)], axis=0)
    mat_aug = jnp.concatenate([mat[0].T, trans[0]], axis=1)
    return _chamfer(fixed_verts, movT_aug, mat_aug)


# speedup vs baseline: 2.0060x; 2.0060x over previous
import functools

import jax
import jax.numpy as jnp
from jax import lax
from jax.experimental import pallas as pl
from jax.experimental.pallas import tpu as pltpu
from jax.experimental.pallas import tpu_sc as plsc

N1 = 8192
N2 = 8192
BM = 2048
CW = 1024


def _chamfer_kernel(fixed_ref, movT_ref, mataug_ref, out_ref,
                    rhs_scr, colmin_scr, rowsum_scr):
    i = pl.program_id(0)
    nsteps = pl.num_programs(0)

    @pl.when(i == 0)
    def _init():
        yT = jnp.dot(mataug_ref[...], movT_ref[...],
                     preferred_element_type=jnp.float32)
        rhs_scr[0:3, :] = yT
        rhs_scr[3:4, :] = jnp.sum(yT * yT, axis=0, keepdims=True)
        rhs_scr[4:5, :] = jnp.ones((1, N2), jnp.float32)
        colmin_scr[...] = jnp.full_like(colmin_scr, jnp.inf)
        rowsum_scr[...] = jnp.zeros_like(rowsum_scr)

    xb = fixed_ref[...]
    x2 = jnp.sum(xb * xb, axis=1, keepdims=True)
    lhs = jnp.concatenate(
        [xb * -2.0, jnp.ones((BM, 1), jnp.float32), x2], axis=1)
    row_min = None
    for c in range(N2 // CW):
        d2 = jnp.dot(lhs, rhs_scr[0:5, c * CW:(c + 1) * CW],
                     preferred_element_type=jnp.float32)
        rm = jnp.min(d2, axis=1)
        row_min = rm if row_min is None else jnp.minimum(row_min, rm)
        col_min = jnp.min(d2, axis=0, keepdims=True)
        colmin_scr[0:1, c * CW:(c + 1) * CW] = jnp.minimum(
            colmin_scr[0:1, c * CW:(c + 1) * CW], col_min)

    row_min = jnp.maximum(row_min, 0.0)
    rowsum_scr[...] += jnp.sum(row_min).reshape(1, 1)

    @pl.when(i == nsteps - 1)
    def _fin():
        col_sum = jnp.sum(jnp.maximum(colmin_scr[...], 0.0))
        out_ref[...] = rowsum_scr[...] / N1 + col_sum.reshape(1, 1) / N2


@jax.jit
def _chamfer(fixed_verts, movT_aug, mat_aug):
    grid = N1 // BM
    out = pl.pallas_call(
        _chamfer_kernel,
        grid=(grid,),
        in_specs=[
            pl.BlockSpec((BM, 3), lambda i: (i, 0)),
            pl.BlockSpec((4, N2), lambda i: (0, 0)),
            pl.BlockSpec((3, 4), lambda i: (0, 0)),
        ],
        out_specs=pl.BlockSpec((1, 1), lambda i: (0, 0)),
        out_shape=jax.ShapeDtypeStruct((1, 1), jnp.float32),
        scratch_shapes=[
            pltpu.VMEM((8, N2), jnp.float32),
            pltpu.VMEM((1, N2), jnp.float32),
            pltpu.VMEM((1, 1), jnp.float32),
        ],
    )(fixed_verts, movT_aug, mat_aug)
    return out[0, 0]


NBINS = 64


def _bin_positions(p, n):
    lo = jnp.min(p)
    hi = jnp.max(p)
    scale = NBINS / (hi - lo + 1e-12)
    b = jnp.clip(jnp.floor((p - lo) * scale), 0, NBINS - 1).astype(jnp.int32)
    onehotT = (jnp.arange(NBINS, dtype=jnp.int32)[:, None] == b[None, :]
               ).astype(jnp.float32)
    cumT = jnp.cumsum(onehotT, axis=1)
    counts = cumT[:, -1]
    offsets = jnp.concatenate(
        [jnp.zeros((1,), jnp.float32), jnp.cumsum(counts)[:-1]])
    pos = jnp.sum(onehotT * (cumT - 1.0 + offsets[:, None]), axis=0)
    return pos.astype(jnp.int32)


_NC = 1
_NS = 16
_NW = _NC * _NS
_BP = N1 // _NW
_IC = 128


def _sc_permute_two(planes_a, pos_a, planes_b, pos_b):
    mesh = plsc.VectorSubcoreMesh(core_axis_name="c", subcore_axis_name="s",
                                  num_cores=_NC)

    @functools.partial(
        pl.kernel, mesh=mesh,
        out_type=[jax.ShapeDtypeStruct((N1,), jnp.float32)] * 6,
        scratch_types=[
            pltpu.VMEM((_BP,), jnp.int32),
            pltpu.VMEM((_BP,), jnp.float32),
            pltpu.VMEM((_BP,), jnp.float32),
            pltpu.VMEM((_BP,), jnp.float32),
            pltpu.VMEM((_BP,), jnp.float32),
            pltpu.VMEM_SHARED((N1,), jnp.float32),
            pltpu.VMEM_SHARED((N1,), jnp.float32),
            pltpu.VMEM_SHARED((N1,), jnp.float32),
        ],
    )
    def k(xa_hbm, ya_hbm, za_hbm, posa_hbm, xb_hbm, yb_hbm, zb_hbm, posb_hbm,
          oxa_hbm, oya_hbm, oza_hbm, oxb_hbm, oyb_hbm, ozb_hbm,
          idx_v, xv, yv, zv, zero_v, x_sh, y_sh, z_sh):
        wid = lax.axis_index("s") * _NC + lax.axis_index("c")
        base = wid * _BP
        win = pl.ds(base, _BP)
        L = 16
        for c in range(_BP // L):
            zero_v[pl.ds(c * L, L)] = jnp.zeros((L,), jnp.float32)

        def permute_one(x_hbm, y_hbm, z_hbm, pos_hbm, ox_hbm, oy_hbm, oz_hbm):
            pltpu.sync_copy(zero_v, x_sh.at[win])
            pltpu.sync_copy(zero_v, y_sh.at[win])
            pltpu.sync_copy(zero_v, z_sh.at[win])
            pltpu.sync_copy(pos_hbm.at[win], idx_v)
            pltpu.sync_copy(x_hbm.at[win], xv)
            pltpu.sync_copy(y_hbm.at[win], yv)
            pltpu.sync_copy(z_hbm.at[win], zv)
            plsc.subcore_barrier()
            pltpu.sync_copy(xv, x_sh.at[idx_v], add=True)
            pltpu.sync_copy(yv, y_sh.at[idx_v], add=True)
            pltpu.sync_copy(zv, z_sh.at[idx_v], add=True)
            plsc.subcore_barrier()
            pltpu.sync_copy(x_sh.at[win], ox_hbm.at[win])
            pltpu.sync_copy(y_sh.at[win], oy_hbm.at[win])
            pltpu.sync_copy(z_sh.at[win], oz_hbm.at[win])

        permute_one(xa_hbm, ya_hbm, za_hbm, posa_hbm, oxa_hbm, oya_hbm,
                    oza_hbm)
        plsc.subcore_barrier()
        permute_one(xb_hbm, yb_hbm, zb_hbm, posb_hbm, oxb_hbm, oyb_hbm,
                    ozb_hbm)

    return k(planes_a[0], planes_a[1], planes_a[2], pos_a,
             planes_b[0], planes_b[1], planes_b[2], pos_b)


def kernel(fixed_verts, moving_verts, mat, trans):
    p_fix = fixed_verts[:, 0]
    p_mov = moving_verts @ mat[0, :, 0]
    pos_f = jnp.arange(N1, dtype=jnp.int32)
    pos_m = jnp.arange(N2, dtype=jnp.int32)
    fx, fy, fz, mx, my, mz = _sc_permute_two(
        (fixed_verts[:, 0], fixed_verts[:, 1], fixed_verts[:, 2]), pos_f,
        (moving_verts[:, 0], moving_verts[:, 1], moving_verts[:, 2]), pos_m)
    fixed_verts = jnp.stack([fx, fy, fz], axis=1)
    movT_aug = jnp.stack(
        [mx, my, mz, jnp.ones((N2,), jnp.float32)], axis=0)
    mat_aug = jnp.concatenate([mat[0].T, trans[0]], axis=1)
    return _chamfer(fixed_verts, movT_aug, mat_aug)


# speedup vs baseline: 2.5367x; 1.2645x over previous
import jax
import jax.numpy as jnp
from jax.experimental import pallas as pl
from jax.experimental.pallas import tpu as pltpu

N1 = 8192
N2 = 8192
BM = 2048
CW = 1024


def _chamfer_kernel(fixed_ref, movT_ref, mataug_ref, out_ref,
                    rhs_scr, colmin_scr, rowsum_scr):
    i = pl.program_id(0)
    nsteps = pl.num_programs(0)

    @pl.when(i == 0)
    def _init():
        yT = jnp.dot(mataug_ref[...], movT_ref[...],
                     preferred_element_type=jnp.float32)
        rhs = jnp.concatenate(
            [yT, jnp.sum(yT * yT, axis=0, keepdims=True),
             jnp.ones((1, N2), jnp.float32)], axis=0)
        rh = rhs.astype(jnp.bfloat16)
        rl = (rhs - rh.astype(jnp.float32)).astype(jnp.bfloat16)
        rhs_scr[0:5, :] = rh
        rhs_scr[5:10, :] = rl
        rhs_scr[10:15, :] = rh
        rhs_scr[15:16, :] = jnp.zeros((1, N2), jnp.bfloat16)
        colmin_scr[...] = jnp.full_like(colmin_scr, jnp.inf)
        rowsum_scr[...] = jnp.zeros_like(rowsum_scr)

    xb = fixed_ref[...]
    x2 = jnp.sum(xb * xb, axis=1, keepdims=True)
    lhs = jnp.concatenate(
        [xb * -2.0, jnp.ones((BM, 1), jnp.float32), x2], axis=1)
    lh = lhs.astype(jnp.bfloat16)
    ll = (lhs - lh.astype(jnp.float32)).astype(jnp.bfloat16)
    lhs16 = jnp.concatenate(
        [lh, lh, ll, jnp.zeros((BM, 1), jnp.bfloat16)], axis=1)
    row_min = None
    for c in range(N2 // CW):
        d2 = jnp.dot(lhs16, rhs_scr[:, c * CW:(c + 1) * CW],
                     preferred_element_type=jnp.float32)
        rm = jnp.min(d2, axis=1)
        row_min = rm if row_min is None else jnp.minimum(row_min, rm)
        col_min = jnp.min(d2, axis=0, keepdims=True)
        colmin_scr[0:1, c * CW:(c + 1) * CW] = jnp.minimum(
            colmin_scr[0:1, c * CW:(c + 1) * CW], col_min)

    row_min = jnp.maximum(row_min, 0.0)
    rowsum_scr[...] += jnp.sum(row_min).reshape(1, 1)

    @pl.when(i == nsteps - 1)
    def _fin():
        col_sum = jnp.sum(jnp.maximum(colmin_scr[...], 0.0))
        out_ref[...] = rowsum_scr[...] / N1 + col_sum.reshape(1, 1) / N2


@jax.jit
def _chamfer(fixed_verts, movT_aug, mat_aug):
    grid = N1 // BM
    out = pl.pallas_call(
        _chamfer_kernel,
        grid=(grid,),
        in_specs=[
            pl.BlockSpec((BM, 3), lambda i: (i, 0)),
            pl.BlockSpec((4, N2), lambda i: (0, 0)),
            pl.BlockSpec((3, 4), lambda i: (0, 0)),
        ],
        out_specs=pl.BlockSpec((1, 1), lambda i: (0, 0)),
        out_shape=jax.ShapeDtypeStruct((1, 1), jnp.float32),
        scratch_shapes=[
            pltpu.VMEM((16, N2), jnp.bfloat16),
            pltpu.VMEM((1, N2), jnp.float32),
            pltpu.VMEM((1, 1), jnp.float32),
        ],
    )(fixed_verts, movT_aug, mat_aug)
    return out[0, 0]


def kernel(fixed_verts, moving_verts, mat, trans):
    movT_aug = jnp.concatenate(
        [moving_verts.T, jnp.ones((1, N2), jnp.float32)], axis=0)
    mat_aug = jnp.concatenate([mat[0].T, trans[0]], axis=1)
    return _chamfer(fixed_verts, movT_aug, mat_aug)


# speedup vs baseline: 2.9012x; 1.1437x over previous
import jax
import jax.numpy as jnp
from jax.experimental import pallas as pl
from jax.experimental.pallas import tpu as pltpu

N1 = 8192
N2 = 8192
BM = 2048
CW = 1024


def _chamfer_kernel(fixed_ref, movT_ref, mataug_ref, out_ref,
                    rhs_scr, colmin_scr, rowsum_scr):
    i = pl.program_id(0)
    nsteps = pl.num_programs(0)

    @pl.when(i == 0)
    def _init():
        yT = jnp.dot(mataug_ref[...], movT_ref[...],
                     preferred_element_type=jnp.float32)
        rhs_scr[0:3, :] = yT
        rhs_scr[3:4, :] = jnp.sum(yT * yT, axis=0, keepdims=True)
        rhs_scr[4:5, :] = jnp.ones((1, N2), jnp.float32)
        colmin_scr[...] = jnp.full_like(colmin_scr, jnp.inf)
        rowsum_scr[...] = jnp.zeros_like(rowsum_scr)

    xb = fixed_ref[...]
    x2 = jnp.sum(xb * xb, axis=1, keepdims=True)
    lhs = jnp.concatenate(
        [xb * -2.0, jnp.ones((BM, 1), jnp.float32), x2], axis=1)
    row_min = None
    nchunk = N2 // CW
    for c in range(nchunk):
        sl = slice(c * CW, (c + 1) * CW)
        if c < nchunk - 1:
            d2 = jnp.dot(lhs, rhs_scr[0:5, sl],
                         preferred_element_type=jnp.float32)
        else:
            d0 = xb[:, 0:1] - rhs_scr[0:1, sl]
            d1 = xb[:, 1:2] - rhs_scr[1:2, sl]
            dz = xb[:, 2:3] - rhs_scr[2:3, sl]
            d2 = d0 * d0 + d1 * d1 + dz * dz
        rm = jnp.min(d2, axis=1)
        row_min = rm if row_min is None else jnp.minimum(row_min, rm)
        col_min = jnp.min(d2, axis=0, keepdims=True)
        colmin_scr[0:1, c * CW:(c + 1) * CW] = jnp.minimum(
            colmin_scr[0:1, c * CW:(c + 1) * CW], col_min)

    row_min = jnp.maximum(row_min, 0.0)
    rowsum_scr[...] += jnp.sum(row_min).reshape(1, 1)

    @pl.when(i == nsteps - 1)
    def _fin():
        col_sum = jnp.sum(jnp.maximum(colmin_scr[...], 0.0))
        out_ref[...] = rowsum_scr[...] / N1 + col_sum.reshape(1, 1) / N2


@jax.jit
def _chamfer(fixed_verts, movT_aug, mat_aug):
    grid = N1 // BM
    out = pl.pallas_call(
        _chamfer_kernel,
        grid=(grid,),
        in_specs=[
            pl.BlockSpec((BM, 3), lambda i: (i, 0)),
            pl.BlockSpec((4, N2), lambda i: (0, 0)),
            pl.BlockSpec((3, 4), lambda i: (0, 0)),
        ],
        out_specs=pl.BlockSpec((1, 1), lambda i: (0, 0)),
        out_shape=jax.ShapeDtypeStruct((1, 1), jnp.float32),
        scratch_shapes=[
            pltpu.VMEM((8, N2), jnp.float32),
            pltpu.VMEM((1, N2), jnp.float32),
            pltpu.VMEM((1, 1), jnp.float32),
        ],
    )(fixed_verts, movT_aug, mat_aug)
    return out[0, 0]


def kernel(fixed_verts, moving_verts, mat, trans):
    movT_aug = jnp.concatenate(
        [moving_verts.T, jnp.ones((1, N2), jnp.float32)], axis=0)
    mat_aug = jnp.concatenate([mat[0].T, trans[0]], axis=1)
    return _chamfer(fixed_verts, movT_aug, mat_aug)


# speedup vs baseline: 3.1342x; 1.0803x over previous
import jax
import jax.numpy as jnp
from jax.experimental import pallas as pl
from jax.experimental.pallas import tpu as pltpu

N1 = 8192
N2 = 8192
BM = 2048
CW = 1024


def _chamfer_kernel(fixed_ref, movT_ref, mataug_ref, out_ref,
                    rhs_scr, colmin_scr, rowsum_scr):
    i = pl.program_id(0)
    nsteps = pl.num_programs(0)

    @pl.when(i == 0)
    def _init():
        yT = jnp.dot(mataug_ref[...], movT_ref[...],
                     preferred_element_type=jnp.float32)
        rhs_scr[0:3, :] = yT
        rhs_scr[3:4, :] = jnp.sum(yT * yT, axis=0, keepdims=True)
        rhs_scr[4:5, :] = jnp.ones((1, N2), jnp.float32)
        colmin_scr[...] = jnp.full_like(colmin_scr, jnp.inf)
        rowsum_scr[...] = jnp.zeros_like(rowsum_scr)

    xb = fixed_ref[...]
    x2 = jnp.sum(xb * xb, axis=1, keepdims=True)
    lhs = jnp.concatenate(
        [xb * -2.0, jnp.ones((BM, 1), jnp.float32), x2], axis=1)
    row_min = None
    for c in range(N2 // CW):
        d2 = jnp.dot(lhs, rhs_scr[0:5, c * CW:(c + 1) * CW],
                     preferred_element_type=jnp.float32)
        rm = jnp.min(d2, axis=1)
        row_min = rm if row_min is None else jnp.minimum(row_min, rm)
        col_min = jnp.min(d2, axis=0, keepdims=True)
        colmin_scr[0:1, c * CW:(c + 1) * CW] = jnp.minimum(
            colmin_scr[0:1, c * CW:(c + 1) * CW], col_min)

    row_min = jnp.maximum(row_min, 0.0)
    rowsum_scr[...] += jnp.sum(row_min).reshape(1, 1)

    @pl.when(i == nsteps - 1)
    def _fin():
        col_sum = jnp.sum(jnp.maximum(colmin_scr[...], 0.0))
        out_ref[...] = rowsum_scr[...] / N1 + col_sum.reshape(1, 1) / N2


@jax.jit
def _chamfer(fixed_verts, movT_aug, mat_aug):
    grid = N1 // BM
    out = pl.pallas_call(
        _chamfer_kernel,
        grid=(grid,),
        in_specs=[
            pl.BlockSpec((BM, 3), lambda i: (i, 0)),
            pl.BlockSpec((4, N2), lambda i: (0, 0)),
            pl.BlockSpec((3, 4), lambda i: (0, 0)),
        ],
        out_specs=pl.BlockSpec((1, 1), lambda i: (0, 0)),
        out_shape=jax.ShapeDtypeStruct((1, 1), jnp.float32),
        scratch_shapes=[
            pltpu.VMEM((8, N2), jnp.float32),
            pltpu.VMEM((1, N2), jnp.float32),
            pltpu.VMEM((1, 1), jnp.float32),
        ],
    )(fixed_verts, movT_aug, mat_aug)
    return out[0, 0]


def kernel(fixed_verts, moving_verts, mat, trans):
    movT_aug = jnp.concatenate(
        [moving_verts.T, jnp.ones((1, N2), jnp.float32)], axis=0)
    mat_aug = jnp.concatenate([mat[0].T, trans[0]], axis=1)
    return _chamfer(fixed_verts, movT_aug, mat_aug)


# speedup vs baseline: 3.1675x; 1.0106x over previous
import jax
import jax.numpy as jnp
from jax.experimental import pallas as pl
from jax.experimental.pallas import tpu as pltpu

N1 = 8192
N2 = 8192
BM = 4096
CW = 1024


def _chamfer_kernel(fixed_ref, movT_ref, mataug_ref, out_ref,
                    rhs_scr, colmin_scr, rowsum_scr):
    i = pl.program_id(0)
    nsteps = pl.num_programs(0)

    @pl.when(i == 0)
    def _init():
        yT = jnp.dot(mataug_ref[...], movT_ref[...],
                     preferred_element_type=jnp.float32)
        rhs_scr[0:3, :] = yT
        rhs_scr[3:4, :] = jnp.sum(yT * yT, axis=0, keepdims=True)
        rhs_scr[4:5, :] = jnp.ones((1, N2), jnp.float32)
        colmin_scr[...] = jnp.full_like(colmin_scr, jnp.inf)
        rowsum_scr[...] = jnp.zeros_like(rowsum_scr)

    xb = fixed_ref[...]
    x2 = jnp.sum(xb * xb, axis=1, keepdims=True)
    lhs = jnp.concatenate(
        [xb * -2.0, jnp.ones((BM, 1), jnp.float32), x2], axis=1)
    row_min = None
    for c in range(N2 // CW):
        d2 = jnp.dot(lhs, rhs_scr[0:5, c * CW:(c + 1) * CW],
                     preferred_element_type=jnp.float32)
        rm = jnp.min(d2, axis=1)
        row_min = rm if row_min is None else jnp.minimum(row_min, rm)
        col_min = jnp.min(d2, axis=0, keepdims=True)
        colmin_scr[0:1, c * CW:(c + 1) * CW] = jnp.minimum(
            colmin_scr[0:1, c * CW:(c + 1) * CW], col_min)

    row_min = jnp.maximum(row_min, 0.0)
    rowsum_scr[...] += jnp.sum(row_min).reshape(1, 1)

    @pl.when(i == nsteps - 1)
    def _fin():
        col_sum = jnp.sum(jnp.maximum(colmin_scr[...], 0.0))
        out_ref[...] = rowsum_scr[...] / N1 + col_sum.reshape(1, 1) / N2


@jax.jit
def _chamfer(fixed_verts, movT_aug, mat_aug):
    grid = N1 // BM
    out = pl.pallas_call(
        _chamfer_kernel,
        grid=(grid,),
        in_specs=[
            pl.BlockSpec((BM, 3), lambda i: (i, 0)),
            pl.BlockSpec((4, N2), lambda i: (0, 0)),
            pl.BlockSpec((3, 4), lambda i: (0, 0)),
        ],
        out_specs=pl.BlockSpec((1, 1), lambda i: (0, 0)),
        out_shape=jax.ShapeDtypeStruct((1, 1), jnp.float32),
        scratch_shapes=[
            pltpu.VMEM((8, N2), jnp.float32),
            pltpu.VMEM((1, N2), jnp.float32),
            pltpu.VMEM((1, 1), jnp.float32),
        ],
    )(fixed_verts, movT_aug, mat_aug)
    return out[0, 0]


def kernel(fixed_verts, moving_verts, mat, trans):
    movT_aug = jnp.concatenate(
        [moving_verts.T, jnp.ones((1, N2), jnp.float32)], axis=0)
    mat_aug = jnp.concatenate([mat[0].T, trans[0]], axis=1)
    return _chamfer(fixed_verts, movT_aug, mat_aug)
